# Initial kernel scaffold; baseline (speedup 1.0000x reference)
import jax
import jax.numpy as jnp
from jax.experimental import pallas as pl


def kernel(x, Wr, br, g1, b1, Wqkv, Wproj, bproj, g2, b2, W1, bf1, W2, bf2, alpha):
    raise NotImplementedError("write your pallas kernel here")


# speedup vs baseline: 1.0257x; 1.0257x over previous
import functools

import jax
import jax.numpy as jnp
from jax import lax
from jax.experimental import pallas as pl

N = 2048
D = 1024
H = 16
DH = 64
E = 8
HID = 4096
CAPN = [512, 409, 307, 204, 204, 163, 143, 106]
NBLK = 256
GRID = N // NBLK

_BS_ITERS = 48


def _router_body(x_ref, wr_ref, br_ref, eidx_ref, rp_ref, tokd_ref):
    x = x_ref[...]
    logits = lax.dot_general(
        x, wr_ref[...], (((1,), (0,)), ((), ())),
        preferred_element_type=jnp.float32,
        precision=lax.Precision.HIGHEST) + br_ref[...]
    m = jnp.max(logits, axis=1, keepdims=True)
    ex = jnp.exp(logits - m)
    probs = ex / jnp.sum(ex, axis=1, keepdims=True)

    avail = jnp.ones((N, 1), jnp.bool_)
    eidx = jnp.zeros((N, 1), jnp.int32)
    for e in reversed(range(E)):
        k = CAPN[e]
        sc = jnp.where(avail, probs[:, e:e + 1], -1e9)

        def bs(_, carry):
            lo, hi = carry
            mid = 0.5 * (lo + hi)
            cnt = jnp.sum((sc >= mid).astype(jnp.int32))
            ge = cnt >= k
            return (jnp.where(ge, mid, lo), jnp.where(ge, hi, mid))

        lo, _ = lax.fori_loop(0, _BS_ITERS, bs,
                              (jnp.float32(-2e9), jnp.float32(1.5)))
        assigned = avail & (sc >= lo)
        eidx = jnp.where(assigned, e, eidx)
        avail = avail & (~assigned)

    rp = jnp.zeros((N, 1), jnp.float32)
    for e in range(E):
        rp = jnp.where(eidx == e, probs[:, e:e + 1], rp)
    eidx_ref[...] = eidx
    rp_ref[...] = rp
    tokd_ref[...] = jnp.left_shift(8, eidx)


def _ln(x, g, b):
    m = jnp.mean(x, axis=1, keepdims=True)
    v = jnp.mean((x - m) ** 2, axis=1, keepdims=True)
    return (x - m) * lax.rsqrt(v + 1e-6) * g + b


def _qkv_body(x_ref, tokd_ref, g1_ref, b1_ref, w_ref, qkv_ref):
    x = x_ref[...]
    tokd = tokd_ref[...]
    xn = _ln(x, g1_ref[...], b1_ref[...])
    cmask = lax.broadcasted_iota(jnp.int32, (1, D), 1) < tokd
    xn = jnp.where(cmask, xn, 0.0)
    qkv = lax.dot_general(xn.astype(jnp.bfloat16), w_ref[...],
                          (((1,), (0,)), ((), ())),
                          preferred_element_type=jnp.float32)
    col = lax.broadcasted_iota(jnp.int32, (1, 3 * D), 1) & (D - 1)
    qkv = jnp.where(col < tokd, qkv, 0.0)
    qkv_ref[...] = qkv.astype(jnp.bfloat16)


def _attn_body(q_ref, k_ref, v_ref, o_ref):
    q = q_ref[0]
    k = k_ref[0]
    s = lax.dot_general(q, k, (((1,), (1,)), ((), ())),
                        preferred_element_type=jnp.float32) * (DH ** -0.5)
    m = jnp.max(s, axis=1, keepdims=True)
    p = jnp.exp(s - m)
    p = p / jnp.sum(p, axis=1, keepdims=True)
    o = lax.dot_general(p.astype(jnp.bfloat16), v_ref[0],
                        (((1,), (0,)), ((), ())),
                        preferred_element_type=jnp.float32)
    o_ref[0] = o.astype(jnp.bfloat16)


def _proj_body(o_ref, x_ref, tokd_ref, w_ref, bp_ref, z_ref):
    o = o_ref[...]
    p = lax.dot_general(o, w_ref[...], (((1,), (0,)), ((), ())),
                        preferred_element_type=jnp.float32) + bp_ref[...]
    cmask = lax.broadcasted_iota(jnp.int32, (1, D), 1) < tokd_ref[...]
    z_ref[...] = x_ref[...] + jnp.where(cmask, p, 0.0)


def _ffn1_body(z_ref, tokd_ref, g2_ref, b2_ref, w1_ref, bf1_ref, h_ref):
    z = z_ref[...]
    tokd = tokd_ref[...]
    zn = _ln(z, g2_ref[...], b2_ref[...])
    cmask = lax.broadcasted_iota(jnp.int32, (1, D), 1) < tokd
    zn = jnp.where(cmask, zn, 0.0)
    h = lax.dot_general(zn.astype(jnp.bfloat16), w1_ref[...],
                        (((1,), (0,)), ((), ())),
                        preferred_element_type=jnp.float32) + bf1_ref[...]
    h = jax.nn.gelu(h)
    hcol = lax.broadcasted_iota(jnp.int32, (1, HID), 1)
    h = jnp.where(hcol < tokd * 4, h, 0.0)
    h_ref[...] = h.astype(jnp.bfloat16)


def _ffn2_body(h_ref, z_ref, rp_ref, tokd_ref, w2_ref, bf2_ref, alpha_ref,
               out_ref):
    h = h_ref[...]
    zp = lax.dot_general(h, w2_ref[...], (((1,), (0,)), ((), ())),
                         preferred_element_type=jnp.float32) + bf2_ref[...]
    cmask = lax.broadcasted_iota(jnp.int32, (1, D), 1) < tokd_ref[...]
    zp = jnp.where(cmask, zp, 0.0)
    gate = alpha_ref[0, 0] * rp_ref[...] + 1.0
    out_ref[...] = z_ref[...] + gate * zp


def _row_spec(w):
    return pl.BlockSpec((NBLK, w), lambda i: (i, 0))


def _full_spec(shape):
    nd = len(shape)
    return pl.BlockSpec(shape, lambda i: (0,) * nd)


def kernel(x, Wr, br, g1, b1, Wqkv, Wproj, bproj, g2, b2, W1, bf1, W2, bf2,
           alpha):
    f32 = jnp.float32
    x2 = x.reshape(N, D)

    eidx, rp, tokd = pl.pallas_call(
        _router_body,
        out_shape=[jax.ShapeDtypeStruct((N, 1), jnp.int32),
                   jax.ShapeDtypeStruct((N, 1), f32),
                   jax.ShapeDtypeStruct((N, 1), jnp.int32)],
    )(x2, Wr, br.reshape(1, E))

    wqkv_b = Wqkv.astype(jnp.bfloat16)
    qkv = pl.pallas_call(
        _qkv_body,
        grid=(GRID,),
        in_specs=[_row_spec(D), _row_spec(1), _full_spec((1, D)),
                  _full_spec((1, D)), _full_spec((D, 3 * D))],
        out_specs=_row_spec(3 * D),
        out_shape=jax.ShapeDtypeStruct((N, 3 * D), jnp.bfloat16),
    )(x2, tokd, g1.reshape(1, D), b1.reshape(1, D), wqkv_b)

    q = qkv[:, 0 * D:1 * D].reshape(N, H, DH).transpose(1, 0, 2)
    k = qkv[:, 1 * D:2 * D].reshape(N, H, DH).transpose(1, 0, 2)
    v = qkv[:, 2 * D:3 * D].reshape(N, H, DH).transpose(1, 0, 2)

    o = pl.pallas_call(
        _attn_body,
        grid=(H, GRID),
        in_specs=[pl.BlockSpec((1, NBLK, DH), lambda h, i: (h, i, 0)),
                  pl.BlockSpec((1, N, DH), lambda h, i: (h, 0, 0)),
                  pl.BlockSpec((1, N, DH), lambda h, i: (h, 0, 0))],
        out_specs=pl.BlockSpec((1, NBLK, DH), lambda h, i: (h, i, 0)),
        out_shape=jax.ShapeDtypeStruct((H, N, DH), jnp.bfloat16),
    )(q, k, v)
    o2 = o.transpose(1, 0, 2).reshape(N, D)

    z = pl.pallas_call(
        _proj_body,
        grid=(GRID,),
        in_specs=[_row_spec(D), _row_spec(D), _row_spec(1),
                  _full_spec((D, D)), _full_spec((1, D))],
        out_specs=_row_spec(D),
        out_shape=jax.ShapeDtypeStruct((N, D), f32),
    )(o2, x2, tokd, Wproj.astype(jnp.bfloat16), bproj.reshape(1, D))

    h = pl.pallas_call(
        _ffn1_body,
        grid=(GRID,),
        in_specs=[_row_spec(D), _row_spec(1), _full_spec((1, D)),
                  _full_spec((1, D)), _full_spec((D, HID)),
                  _full_spec((1, HID))],
        out_specs=_row_spec(HID),
        out_shape=jax.ShapeDtypeStruct((N, HID), jnp.bfloat16),
    )(z, tokd, g2.reshape(1, D), b2.reshape(1, D), W1.astype(jnp.bfloat16),
      bf1.reshape(1, HID))

    out = pl.pallas_call(
        _ffn2_body,
        grid=(GRID,),
        in_specs=[_row_spec(HID), _row_spec(D), _row_spec(1), _row_spec(1),
                  _full_spec((HID, D)), _full_spec((1, D)),
                  _full_spec((1, 1))],
        out_specs=_row_spec(D),
        out_shape=jax.ShapeDtypeStruct((N, D), f32),
    )(h, z, rp, tokd, W2.astype(jnp.bfloat16), bf2.reshape(1, D),
      alpha.reshape(1, 1))

    return out.reshape(1, N, D)


# speedup vs baseline: 1.1075x; 1.0798x over previous
import functools

import numpy as np
import jax
import jax.numpy as jnp
from jax import lax
from jax.experimental import pallas as pl
from jax.experimental.pallas import tpu as pltpu
from jax.experimental.pallas import tpu_sc as plsc

N = 2048
D = 1024
H = 16
DH = 64
E = 8
CAPN = [512, 409, 307, 204, 204, 163, 143, 106]

_ORDER = list(range(E - 1, -1, -1))
_SIZES = [CAPN[e] for e in _ORDER]
_OFFS = np.concatenate([[0], np.cumsum(_SIZES)])
OFF_OF_EXPERT = {e: int(_OFFS[i]) for i, e in enumerate(_ORDER)}

BUCKETS = [
    (0, 106, 1024),
    (106, 143, 512),
    (249, 163, 256),
    (412, 204, 128),
    (616, 1432, 64),
]
TOKD_SORTED = np.concatenate([
    np.full(CAPN[e], 8 << e, np.int32) for e in _ORDER])

HEAD_GROUPS = [
    (0, 1, 2048, 2048),
    (1, 1, 616, 640),
    (2, 2, 412, 512),
    (4, 4, 249, 256),
    (8, 8, 106, 128),
]

_BS_ITERS = 48
NBLK = 256
GRID = N // NBLK


def _pad_rows(a, m):
    return jnp.pad(a, ((0, m - a.shape[0]),) + ((0, 0),) * (a.ndim - 1))


def _router_body(x_ref, wr_ref, br_ref, rp_ref, eidx_ref):
    x = x_ref[...]
    logits = lax.dot_general(
        x, wr_ref[...], (((1,), (0,)), ((), ())),
        preferred_element_type=jnp.float32,
        precision=lax.Precision.HIGHEST) + br_ref[...]
    m = jnp.max(logits, axis=1, keepdims=True)
    ex = jnp.exp(logits - m)
    probs = ex / jnp.sum(ex, axis=1, keepdims=True)

    avail = jnp.ones((N, 1), jnp.bool_)
    eidx = jnp.zeros((N, 1), jnp.int32)
    for e in reversed(range(E)):
        k = CAPN[e]
        sc = jnp.where(avail, probs[:, e:e + 1], -1e9)

        def bs(_, carry):
            lo, hi = carry
            mid = 0.5 * (lo + hi)
            cnt = jnp.sum((sc >= mid).astype(jnp.int32))
            ge = cnt >= k
            return (jnp.where(ge, mid, lo), jnp.where(ge, hi, mid))

        lo, _ = lax.fori_loop(0, _BS_ITERS, bs,
                              (jnp.float32(-2e9), jnp.float32(1.5)))
        assigned = avail & (sc >= lo)
        eidx = jnp.where(assigned, e, eidx)
        avail = avail & (~assigned)

    rp = jnp.zeros((N, 1), jnp.float32)
    for e in range(E):
        rp = jnp.where(eidx == e, probs[:, e:e + 1], rp)
    rp_ref[...] = rp
    eidx_ref[...] = eidx


def _slot_body(eidx_full_ref, eidx_blk_ref, slot_ref):
    i = pl.program_id(0)
    oh = (eidx_full_ref[...]
          == lax.broadcasted_iota(jnp.int32, (1, E), 1)).astype(jnp.bfloat16)
    row = lax.broadcasted_iota(jnp.int32, (NBLK, N), 0) + i * NBLK
    tl = (lax.broadcasted_iota(jnp.int32, (NBLK, N), 1)
          < row).astype(jnp.bfloat16)
    cnt = lax.dot_general(tl, oh, (((1,), (0,)), ((), ())),
                          preferred_element_type=jnp.float32)
    eb = eidx_blk_ref[...]
    slot = jnp.zeros((NBLK, 1), jnp.int32)
    for e in range(E):
        slot = jnp.where(
            eb == e,
            OFF_OF_EXPERT[e] + cnt[:, e:e + 1].astype(jnp.int32), slot)
    slot_ref[...] = slot


def _perm_body(slot_ref, perm_ref):
    i = pl.program_id(0)
    s = slot_ref[...]
    cols = lax.broadcasted_iota(jnp.int32, (1, NBLK), 1) + i * NBLK
    oh = (s == cols).astype(jnp.float32)
    tvec = lax.broadcasted_iota(jnp.int32, (1, N), 1).astype(jnp.float32)
    pm = lax.dot_general(tvec, oh, (((1,), (0,)), ((), ())),
                         preferred_element_type=jnp.float32,
                         precision=lax.Precision.HIGHEST)
    perm_ref[...] = pm.astype(jnp.int32)


def _sc_gather(src, idx, width):
    nw = 32
    chunk = N // nw
    mesh = plsc.VectorSubcoreMesh(core_axis_name="c", subcore_axis_name="s")

    @functools.partial(
        pl.kernel, mesh=mesh,
        out_type=jax.ShapeDtypeStruct((N, width), jnp.float32),
        scratch_types=[pltpu.VMEM((chunk,), jnp.int32),
                       pltpu.VMEM((chunk, width), jnp.float32),
                       pltpu.SemaphoreType.DMA],
    )
    def g(src_hbm, idx_hbm, out_hbm, idx_v, rows_v, sem):
        wid = lax.axis_index("s") * 2 + lax.axis_index("c")
        base = wid * chunk
        pltpu.sync_copy(idx_hbm.at[pl.ds(base, chunk)], idx_v)
        pltpu.async_copy(src_hbm.at[idx_v], rows_v, sem).wait()
        pltpu.sync_copy(rows_v, out_hbm.at[pl.ds(base, chunk)])

    return g(src, idx)


def _ln(x, g, b):
    m = jnp.mean(x, axis=1, keepdims=True)
    v = jnp.mean((x - m) ** 2, axis=1, keepdims=True)
    return (x - m) * lax.rsqrt(v + 1e-6) * g + b


def _qkv_bucket_body(dim, mixed, *refs):
    if mixed:
        x_ref, tokd_ref, g1_ref, b1_ref, w_ref, out_ref = refs
        tokd = tokd_ref[...]
    else:
        x_ref, g1_ref, b1_ref, w_ref, out_ref = refs
    xn = _ln(x_ref[...], g1_ref[...], b1_ref[...])
    if mixed:
        cm = lax.broadcasted_iota(jnp.int32, (1, D), 1) < tokd
        xn = jnp.where(cm, xn, 0.0)
    xt = xn[:, :dim].astype(jnp.bfloat16)
    qkv = lax.dot_general(xt, w_ref[...], (((1,), (0,)), ((), ())),
                          preferred_element_type=jnp.float32)
    if mixed:
        col = lax.broadcasted_iota(jnp.int32, (1, 3 * dim), 1) & (dim - 1)
        qkv = jnp.where(col < tokd, qkv, 0.0)
    out_ref[...] = qkv.astype(jnp.bfloat16)


def _attn_group_body(nk_pad, q_ref, k_ref, v_ref, o_ref):
    q = q_ref[0]
    k = k_ref[0]
    s = lax.dot_general(q, k, (((1,), (1,)), ((), ())),
                        preferred_element_type=jnp.float32) * (DH ** -0.5)
    m = jnp.max(s, axis=1, keepdims=True)
    if nk_pad < N:
        m = jnp.maximum(m, 0.0)
    p = jnp.exp(s - m)
    denom = jnp.sum(p, axis=1, keepdims=True)
    if nk_pad < N:
        denom = denom + (N - nk_pad) * jnp.exp(-m)
    o = lax.dot_general((p / denom).astype(jnp.bfloat16), v_ref[0],
                        (((1,), (0,)), ((), ())),
                        preferred_element_type=jnp.float32)
    o_ref[0] = o.astype(jnp.bfloat16)


def _tail_bucket_body(dim, mixed, *refs):
    if mixed:
        (o_ref, x_ref, tokd_ref, wp_ref, bp_ref, g2_ref, b2_ref,
         w1_ref, bf1_ref, w2_ref, bf2_ref, delta_ref, zp_ref) = refs
        tokd = tokd_ref[...]
    else:
        (o_ref, x_ref, wp_ref, bp_ref, g2_ref, b2_ref,
         w1_ref, bf1_ref, w2_ref, bf2_ref, delta_ref, zp_ref) = refs
    nrow = x_ref.shape[0]
    pr = lax.dot_general(o_ref[...], wp_ref[...], (((1,), (0,)), ((), ())),
                         preferred_element_type=jnp.float32) + bp_ref[...]
    if mixed:
        cm = lax.broadcasted_iota(jnp.int32, (1, dim), 1) < tokd
        pr = jnp.where(cm, pr, 0.0)
    if dim < D:
        z = x_ref[...] + jnp.concatenate(
            [pr, jnp.zeros((nrow, D - dim), jnp.float32)], axis=1)
    else:
        z = x_ref[...] + pr
    zn = _ln(z, g2_ref[...], b2_ref[...])[:, :dim]
    if mixed:
        zn = jnp.where(cm, zn, 0.0)
    h = lax.dot_general(zn.astype(jnp.bfloat16), w1_ref[...],
                        (((1,), (0,)), ((), ())),
                        preferred_element_type=jnp.float32) + bf1_ref[...]
    h = jax.nn.gelu(h)
    if mixed:
        hcol = lax.broadcasted_iota(jnp.int32, (1, 4 * dim), 1)
        h = jnp.where(hcol < tokd * 4, h, 0.0)
    zp = lax.dot_general(h.astype(jnp.bfloat16), w2_ref[...],
                         (((1,), (0,)), ((), ())),
                         preferred_element_type=jnp.float32) + bf2_ref[...]
    if mixed:
        zp = jnp.where(cm, zp, 0.0)
    delta_ref[...] = pr
    zp_ref[...] = zp


def _combine_body(x_ref, delta_ref, zp_ref, rp_ref, alpha_ref, out_ref):
    gate = alpha_ref[0, 0] * rp_ref[...] + 1.0
    out_ref[...] = x_ref[...] + delta_ref[...] + gate * zp_ref[...]


def _full_spec(shape):
    nd = len(shape)
    return pl.BlockSpec(shape, lambda *a: (0,) * nd)


def kernel(x, Wr, br, g1, b1, Wqkv, Wproj, bproj, g2, b2, W1, bf1, W2, bf2,
           alpha):
    f32 = jnp.float32
    bf = jnp.bfloat16
    x2 = x.reshape(N, D)

    rp, eidx = pl.pallas_call(
        _router_body,
        out_shape=[jax.ShapeDtypeStruct((N, 1), f32),
                   jax.ShapeDtypeStruct((N, 1), jnp.int32)],
    )(x2, Wr, br.reshape(1, E))

    slot = pl.pallas_call(
        _slot_body,
        grid=(GRID,),
        in_specs=[_full_spec((N, 1)),
                  pl.BlockSpec((NBLK, 1), lambda i: (i, 0))],
        out_specs=pl.BlockSpec((NBLK, 1), lambda i: (i, 0)),
        out_shape=jax.ShapeDtypeStruct((N, 1), jnp.int32),
    )(eidx, eidx)

    perm = pl.pallas_call(
        _perm_body,
        grid=(GRID,),
        in_specs=[_full_spec((N, 1))],
        out_specs=pl.BlockSpec((1, NBLK), lambda i: (0, i)),
        out_shape=jax.ShapeDtypeStruct((1, N), jnp.int32),
    )(slot).reshape(N)

    xs = _sc_gather(x2, perm, D)

    g1r, b1r = g1.reshape(1, D), b1.reshape(1, D)
    g2r, b2r = g2.reshape(1, D), b2.reshape(1, D)
    tokd_s = jnp.asarray(TOKD_SORTED.reshape(N, 1))

    qkv_parts = []
    for start, n, dim in BUCKETS:
        mixed = dim == 64 and n > 512
        npad = -(-n // 8) * 8
        xb = _pad_rows(lax.slice(xs, (start, 0), (start + n, D)), npad)
        wb = jnp.concatenate(
            [Wqkv[:dim, j * D:j * D + dim] for j in range(3)],
            axis=1).astype(bf)
        ins = [xb]
        specs = [_full_spec((npad, D))]
        if mixed:
            tb = _pad_rows(lax.slice(tokd_s, (start, 0), (start + n, 1)),
                           npad)
            ins.append(tb)
            specs.append(_full_spec((npad, 1)))
        ins += [g1r, b1r, wb]
        specs += [_full_spec((1, D)), _full_spec((1, D)),
                  _full_spec((dim, 3 * dim))]
        qkv_b = pl.pallas_call(
            functools.partial(_qkv_bucket_body, dim, mixed),
            in_specs=specs,
            out_specs=_full_spec((npad, 3 * dim)),
            out_shape=jax.ShapeDtypeStruct((npad, 3 * dim), bf),
        )(*ins)
        qkv_parts.append(qkv_b[:n])

    def head_qkv(h, part):
        segs = []
        for (start, n, dim), qkv_b in zip(BUCKETS, qkv_parts):
            if dim > h * DH:
                off = part * dim + h * DH
                segs.append(qkv_b[:, off:off + DH])
        return jnp.concatenate(segs, axis=0)

    o_heads = {}
    for h0, nh, nq, nq_pad in HEAD_GROUPS:
        qg = jnp.stack([_pad_rows(head_qkv(h0 + i, 0), nq_pad)
                        for i in range(nh)])
        kg = jnp.stack([_pad_rows(head_qkv(h0 + i, 1), nq_pad)
                        for i in range(nh)])
        vg = jnp.stack([_pad_rows(head_qkv(h0 + i, 2), nq_pad)
                        for i in range(nh)])
        if nq_pad == N:
            og = pl.pallas_call(
                functools.partial(_attn_group_body, nq_pad),
                grid=(nh, GRID),
                in_specs=[pl.BlockSpec((1, NBLK, DH), lambda h, i: (h, i, 0)),
                          pl.BlockSpec((1, N, DH), lambda h, i: (h, 0, 0)),
                          pl.BlockSpec((1, N, DH), lambda h, i: (h, 0, 0))],
                out_specs=pl.BlockSpec((1, NBLK, DH), lambda h, i: (h, i, 0)),
                out_shape=jax.ShapeDtypeStruct((nh, nq_pad, DH), bf),
            )(qg, kg, vg)
        else:
            og = pl.pallas_call(
                functools.partial(_attn_group_body, nq_pad),
                grid=(nh,),
                in_specs=[pl.BlockSpec((1, nq_pad, DH), lambda h: (h, 0, 0))] * 3,
                out_specs=pl.BlockSpec((1, nq_pad, DH), lambda h: (h, 0, 0)),
                out_shape=jax.ShapeDtypeStruct((nh, nq_pad, DH), bf),
            )(qg, kg, vg)
        for i in range(nh):
            o_heads[h0 + i] = og[i]

    delta_parts, zp_parts = [], []
    for start, n, dim in BUCKETS:
        mixed = dim == 64 and n > 512
        npad = -(-n // 8) * 8
        nheads = dim // DH
        ob = jnp.concatenate(
            [o_heads[h][start:start + n] for h in range(nheads)], axis=1)
        ob = _pad_rows(ob, npad)
        xb = _pad_rows(lax.slice(xs, (start, 0), (start + n, D)), npad)
        ins = [ob, xb]
        specs = [_full_spec((npad, dim)), _full_spec((npad, D))]
        if mixed:
            tb = _pad_rows(lax.slice(tokd_s, (start, 0), (start + n, 1)),
                           npad)
            ins.append(tb)
            specs.append(_full_spec((npad, 1)))
        wp = Wproj[:dim, :dim].astype(bf)
        w1 = W1[:dim, :4 * dim].astype(bf)
        w2 = W2[:4 * dim, :dim].astype(bf)
        ins += [wp, bproj[:dim].reshape(1, dim), g2r, b2r,
                w1, bf1[:4 * dim].reshape(1, 4 * dim),
                w2, bf2[:dim].reshape(1, dim)]
        specs += [_full_spec((dim, dim)), _full_spec((1, dim)),
                  _full_spec((1, D)), _full_spec((1, D)),
                  _full_spec((dim, 4 * dim)), _full_spec((1, 4 * dim)),
                  _full_spec((4 * dim, dim)), _full_spec((1, dim))]
        delta_b, zp_b = pl.pallas_call(
            functools.partial(_tail_bucket_body, dim, mixed),
            in_specs=specs,
            out_specs=[_full_spec((npad, dim)), _full_spec((npad, dim))],
            out_shape=[jax.ShapeDtypeStruct((npad, dim), f32),
                       jax.ShapeDtypeStruct((npad, dim), f32)],
        )(*ins)
        delta_parts.append(jnp.pad(delta_b[:n], ((0, 0), (0, D - dim))))
        zp_parts.append(jnp.pad(zp_b[:n], ((0, 0), (0, D - dim))))

    delta_s = jnp.concatenate(delta_parts, axis=0)
    zp_s = jnp.concatenate(zp_parts, axis=0)

    slot1 = slot.reshape(N)
    delta = _sc_gather(delta_s, slot1, D)
    zp = _sc_gather(zp_s, slot1, D)

    out = pl.pallas_call(
        _combine_body,
        grid=(GRID,),
        in_specs=[pl.BlockSpec((NBLK, D), lambda i: (i, 0))] * 3
        + [pl.BlockSpec((NBLK, 1), lambda i: (i, 0)), _full_spec((1, 1))],
        out_specs=pl.BlockSpec((NBLK, D), lambda i: (i, 0)),
        out_shape=jax.ShapeDtypeStruct((N, D), f32),
    )(x2, delta, zp, rp, alpha.reshape(1, 1))

    return out.reshape(1, N, D)


# speedup vs baseline: 1.9272x; 1.7402x over previous
import functools

import numpy as np
import jax
import jax.numpy as jnp
from jax import lax
from jax.experimental import pallas as pl
from jax.experimental.pallas import tpu as pltpu
from jax.experimental.pallas import tpu_sc as plsc

N = 2048
D = 1024
H = 16
DH = 64
E = 8
CAPN = [512, 409, 307, 204, 204, 163, 143, 106]

_ORDER = list(range(E - 1, -1, -1))
_SIZES = [CAPN[e] for e in _ORDER]
_OFFS = np.concatenate([[0], np.cumsum(_SIZES)])
OFF_OF_EXPERT = {e: int(_OFFS[i]) for i, e in enumerate(_ORDER)}

BUCKETS = [
    (0, 106, 1024),
    (106, 143, 512),
    (249, 163, 256),
    (412, 204, 128),
    (616, 1432, 64),
]
TOKD_SORTED = np.concatenate(
    [np.full(CAPN[e], 8 << e, np.int32) for e in _ORDER])

HEAD_GROUPS = [
    (0, 1, 2048),
    (1, 1, 616),
    (2, 2, 412),
    (4, 4, 249),
    (8, 8, 106),
]

_BS_ITERS = 48
NBLK = 256
GRID = N // NBLK


def _router_body(x_ref, wr_ref, br_ref, rp_ref, slot_ref):
    x = x_ref[...]
    logits = lax.dot_general(
        x, wr_ref[...], (((1,), (0,)), ((), ())),
        preferred_element_type=jnp.float32,
        precision=lax.Precision.HIGHEST) + br_ref[...]
    m = jnp.max(logits, axis=1, keepdims=True)
    ex = jnp.exp(logits - m)
    probs = ex / jnp.sum(ex, axis=1, keepdims=True)

    avail = jnp.ones((N, 1), jnp.bool_)
    eidx = jnp.zeros((N, 1), jnp.int32)
    for e in reversed(range(E)):
        k = CAPN[e]
        sc = jnp.where(avail, probs[:, e:e + 1], -1e9)

        def bs(_, carry):
            lo, hi = carry
            mid = 0.5 * (lo + hi)
            cnt = jnp.sum((sc >= mid).astype(jnp.int32))
            ge = cnt >= k
            return (jnp.where(ge, mid, lo), jnp.where(ge, hi, mid))

        lo, _ = lax.fori_loop(0, _BS_ITERS, bs,
                              (jnp.float32(-2e9), jnp.float32(1.5)))
        assigned = avail & (sc >= lo)
        eidx = jnp.where(assigned, e, eidx)
        avail = avail & (~assigned)

    rp = jnp.zeros((N, 1), jnp.float32)
    for e in range(E):
        rp = jnp.where(eidx == e, probs[:, e:e + 1], rp)
    rp_ref[...] = rp

    oh = (eidx == lax.broadcasted_iota(jnp.int32, (1, E), 1)
          ).astype(jnp.bfloat16)
    tl = (lax.broadcasted_iota(jnp.int32, (N, N), 1)
          < lax.broadcasted_iota(jnp.int32, (N, N), 0)
          ).astype(jnp.bfloat16)
    cnt = lax.dot_general(tl, oh, (((1,), (0,)), ((), ())),
                          preferred_element_type=jnp.float32)
    slot = jnp.zeros((N, 1), jnp.int32)
    for e in range(E):
        slot = jnp.where(
            eidx == e,
            OFF_OF_EXPERT[e] + cnt[:, e:e + 1].astype(jnp.int32), slot)
    slot_ref[...] = slot


def _sc_scatter_rows(src, idx, width):
    nw = 32
    chunk = N // nw
    mesh = plsc.VectorSubcoreMesh(core_axis_name="c", subcore_axis_name="s")

    @functools.partial(
        pl.kernel, mesh=mesh,
        out_type=jax.ShapeDtypeStruct((N, width), jnp.float32),
        scratch_types=[pltpu.VMEM((chunk,), jnp.int32),
                       pltpu.VMEM((chunk, width), jnp.float32),
                       pltpu.SemaphoreType.DMA],
    )
    def g(src_hbm, idx_hbm, out_hbm, idx_v, rows_v, sem):
        wid = lax.axis_index("s") * 2 + lax.axis_index("c")
        base = wid * chunk
        pltpu.sync_copy(idx_hbm.at[pl.ds(base, chunk)], idx_v)
        pltpu.sync_copy(src_hbm.at[pl.ds(base, chunk)], rows_v)
        pltpu.async_copy(rows_v, out_hbm.at[idx_v], sem).wait()

    return g(src, idx)


def _ln(x, g, b):
    m = jnp.mean(x, axis=1, keepdims=True)
    v = jnp.mean((x - m) ** 2, axis=1, keepdims=True)
    return (x - m) * lax.rsqrt(v + 1e-6) * g + b


def _npad(n):
    return -(-n // 8) * 8


def _mixed_tokd(n, start):
    s = lax.broadcasted_iota(jnp.int32, (n, 1), 0) + start
    tokd = jnp.full((n, 1), 8, jnp.int32)
    for e in (1, 2, 3):
        lo = OFF_OF_EXPERT[e]
        tokd = jnp.where(s < lo + CAPN[e],
                         jnp.where(s >= lo, 8 << e, tokd), tokd)
    return tokd


def _fwd_body(xs_ref, g1_ref, b1_ref, wqkv_ref, *o_refs):
    g1 = g1_ref[...]
    b1 = b1_ref[...]
    qkv_parts = []
    for start, n, dim in BUCKETS:
        mixed = dim == 64 and n > 512
        xb = xs_ref[start:start + n, :]
        xn = _ln(xb, g1, b1)
        if mixed:
            tokd = _mixed_tokd(n, start)
            cm = lax.broadcasted_iota(jnp.int32, (1, D), 1) < tokd
            xn = jnp.where(cm, xn, 0.0)
        xt = xn[:, :dim].astype(jnp.bfloat16)
        wb = jnp.concatenate(
            [wqkv_ref[:dim, j * D:j * D + dim] for j in range(3)], axis=1)
        qkv = lax.dot_general(xt, wb, (((1,), (0,)), ((), ())),
                              preferred_element_type=jnp.float32)
        if mixed:
            col = lax.broadcasted_iota(jnp.int32, (1, 3 * dim), 1) & (dim - 1)
            qkv = jnp.where(col < tokd, qkv, 0.0)
        qkv_parts.append(qkv.astype(jnp.bfloat16))

    def head_qkv(h, part):
        segs = []
        for (start, n, dim), qkv_b in zip(BUCKETS, qkv_parts):
            if dim > h * DH:
                off = part * dim + h * DH
                segs.append(qkv_b[:, off:off + DH])
        return jnp.concatenate(segs, axis=0)

    o_heads = {}
    for h0, nh, nq in HEAD_GROUPS:
        for i in range(nh):
            h = h0 + i
            q = head_qkv(h, 0)
            k = head_qkv(h, 1)
            v = head_qkv(h, 2)
            if nq == N:
                chunks = []
                for c in range(GRID):
                    qc = q[c * NBLK:(c + 1) * NBLK, :]
                    s = lax.dot_general(
                        qc, k, (((1,), (1,)), ((), ())),
                        preferred_element_type=jnp.float32) * (DH ** -0.5)
                    mx = jnp.max(s, axis=1, keepdims=True)
                    p = jnp.exp(s - mx)
                    p = p / jnp.sum(p, axis=1, keepdims=True)
                    chunks.append(lax.dot_general(
                        p.astype(jnp.bfloat16), v, (((1,), (0,)), ((), ())),
                        preferred_element_type=jnp.float32
                    ).astype(jnp.bfloat16))
                o_heads[h] = jnp.concatenate(chunks, axis=0)
            else:
                s = lax.dot_general(
                    q, k, (((1,), (1,)), ((), ())),
                    preferred_element_type=jnp.float32) * (DH ** -0.5)
                mx = jnp.maximum(jnp.max(s, axis=1, keepdims=True), 0.0)
                p = jnp.exp(s - mx)
                denom = (jnp.sum(p, axis=1, keepdims=True)
                         + (N - nq) * jnp.exp(-mx))
                o_heads[h] = lax.dot_general(
                    (p / denom).astype(jnp.bfloat16), v,
                    (((1,), (0,)), ((), ())),
                    preferred_element_type=jnp.float32).astype(jnp.bfloat16)

    for bi, (start, n, dim) in enumerate(BUCKETS):
        ob = jnp.concatenate(
            [o_heads[h][start:start + n, :] for h in range(dim // DH)],
            axis=1)
        o_refs[bi][...] = jnp.pad(ob, ((0, _npad(n) - n), (0, 0)))


def _tail_body(x_ref, xs_ref, slot_ref, rp_ref, wp_ref, bp_ref, g2_ref,
               b2_ref, w1_ref, bf1_ref, w2_ref, bf2_ref, alpha_ref,
               o0_ref, o1_ref, o2_ref, o3_ref, o4_ref, out_ref):
    o_refs = [o0_ref, o1_ref, o2_ref, o3_ref, o4_ref]
    g2 = g2_ref[...]
    b2 = b2_ref[...]
    delta_parts, zp_parts = [], []
    for bi, (start, n, dim) in enumerate(BUCKETS):
        mixed = dim == 64 and n > 512
        ob = o_refs[bi][0:n, :]
        xb = xs_ref[start:start + n, :]
        pr = lax.dot_general(ob, wp_ref[:dim, :dim],
                             (((1,), (0,)), ((), ())),
                             preferred_element_type=jnp.float32
                             ) + bp_ref[:, :dim]
        if mixed:
            tokd = _mixed_tokd(n, start)
            cm = lax.broadcasted_iota(jnp.int32, (1, dim), 1) < tokd
            pr = jnp.where(cm, pr, 0.0)
        if dim < D:
            z = xb + jnp.concatenate(
                [pr, jnp.zeros((n, D - dim), jnp.float32)], axis=1)
        else:
            z = xb + pr
        zn = _ln(z, g2, b2)[:, :dim]
        if mixed:
            zn = jnp.where(cm, zn, 0.0)
        h = lax.dot_general(zn.astype(jnp.bfloat16), w1_ref[:dim, :4 * dim],
                            (((1,), (0,)), ((), ())),
                            preferred_element_type=jnp.float32
                            ) + bf1_ref[:, :4 * dim]
        h = jax.nn.gelu(h)
        if mixed:
            hcol = lax.broadcasted_iota(jnp.int32, (1, 4 * dim), 1)
            h = jnp.where(hcol < tokd * 4, h, 0.0)
        zp = lax.dot_general(h.astype(jnp.bfloat16), w2_ref[:4 * dim, :dim],
                             (((1,), (0,)), ((), ())),
                             preferred_element_type=jnp.float32
                             ) + bf2_ref[:, :dim]
        if mixed:
            zp = jnp.where(cm, zp, 0.0)
        pad = ((0, 0), (0, D - dim))
        delta_parts.append(jnp.pad(pr, pad).astype(jnp.bfloat16))
        zp_parts.append(jnp.pad(zp, pad).astype(jnp.bfloat16))

    delta_s = jnp.concatenate(delta_parts, axis=0)
    zp_s = jnp.concatenate(zp_parts, axis=0)
    alpha = alpha_ref[0, 0]
    for c in range(GRID):
        sl = slot_ref[c * NBLK:(c + 1) * NBLK, :]
        sel = (sl == lax.broadcasted_iota(jnp.int32, (1, N), 1)
               ).astype(jnp.bfloat16)
        delta = lax.dot_general(sel, delta_s, (((1,), (0,)), ((), ())),
                                preferred_element_type=jnp.float32)
        zp = lax.dot_general(sel, zp_s, (((1,), (0,)), ((), ())),
                             preferred_element_type=jnp.float32)
        gate = alpha * rp_ref[c * NBLK:(c + 1) * NBLK, :] + 1.0
        out_ref[c * NBLK:(c + 1) * NBLK, :] = (
            x_ref[c * NBLK:(c + 1) * NBLK, :] + delta + gate * zp)


def _full_spec(shape):
    nd = len(shape)
    return pl.BlockSpec(shape, lambda *a: (0,) * nd)


def kernel(x, Wr, br, g1, b1, Wqkv, Wproj, bproj, g2, b2, W1, bf1, W2, bf2,
           alpha):
    f32 = jnp.float32
    bf = jnp.bfloat16
    x2 = x.reshape(N, D)

    rp, slot = pl.pallas_call(
        _router_body,
        out_shape=[jax.ShapeDtypeStruct((N, 1), f32),
                   jax.ShapeDtypeStruct((N, 1), jnp.int32)],
    )(x2, Wr, br.reshape(1, E))

    xs = _sc_scatter_rows(x2, slot.reshape(N), D)

    o_buckets = pl.pallas_call(
        _fwd_body,
        out_shape=[jax.ShapeDtypeStruct((_npad(n), dim), bf)
                   for _, n, dim in BUCKETS],
    )(xs, g1.reshape(1, D), b1.reshape(1, D), Wqkv.astype(bf))

    out = pl.pallas_call(
        _tail_body,
        out_shape=jax.ShapeDtypeStruct((N, D), f32),
    )(x2, xs, slot, rp, Wproj.astype(bf), bproj.reshape(1, D),
      g2.reshape(1, D), b2.reshape(1, D), W1.astype(bf),
      bf1.reshape(1, 4 * D), W2.astype(bf), bf2.reshape(1, D),
      alpha.reshape(1, 1), *o_buckets)

    return out.reshape(1, N, D)


# speedup vs baseline: 2.9793x; 1.5459x over previous
import functools

import numpy as np
import jax
import jax.numpy as jnp
from jax import lax
from jax.experimental import pallas as pl

N = 2048
D = 1024
H = 16
DH = 64
E = 8
CAPN = [512, 409, 307, 204, 204, 163, 143, 106]

_ORDER = list(range(E - 1, -1, -1))
_SIZES = [CAPN[e] for e in _ORDER]
_OFFS = np.concatenate([[0], np.cumsum(_SIZES)])
OFF_OF_EXPERT = {e: int(_OFFS[i]) for i, e in enumerate(_ORDER)}

BUCKETS = [
    (0, 106, 1024),
    (106, 143, 512),
    (249, 163, 256),
    (412, 204, 128),
    (616, 1432, 64),
]

HEAD_GROUPS = [
    (0, 1, 2048),
    (1, 1, 616),
    (2, 2, 412),
    (4, 4, 249),
    (8, 8, 106),
]

_SEARCH_ITERS = 12
_NSPLIT = 16
NBLK = 256
GRID = N // NBLK


def _npad(n):
    return -(-n // 8) * 8


def _mixed_tokd(n, start):
    s = lax.broadcasted_iota(jnp.int32, (n, 1), 0) + start
    tokd = jnp.full((n, 1), 8, jnp.int32)
    for e in (1, 2, 3):
        lo = OFF_OF_EXPERT[e]
        tokd = jnp.where(s < lo + CAPN[e],
                         jnp.where(s >= lo, 8 << e, tokd), tokd)
    return tokd


def _ln(x, g, b):
    m = jnp.mean(x, axis=1, keepdims=True)
    v = jnp.mean((x - m) ** 2, axis=1, keepdims=True)
    return (x - m) * lax.rsqrt(v + 1e-6) * g + b


def _fwd_body(x_ref, wr_ref, br_ref, g1_ref, b1_ref, wqkv_ref,
              xs_ref, slot_out_ref, rps_ref, *o_refs):
    x = x_ref[...]
    logits = lax.dot_general(
        x, wr_ref[...], (((1,), (0,)), ((), ())),
        preferred_element_type=jnp.float32,
        precision=lax.Precision.HIGHEST) + br_ref[...]
    mx = jnp.max(logits, axis=1, keepdims=True)
    ex = jnp.exp(logits - mx)
    probs = ex / jnp.sum(ex, axis=1, keepdims=True)

    frac = ((lax.broadcasted_iota(jnp.int32, (1, _NSPLIT), 1)
             .astype(jnp.float32) + 1.0) / (_NSPLIT + 1.0))
    avail = jnp.ones((N, 1), jnp.bool_)
    eidx = jnp.zeros((N, 1), jnp.int32)
    for e in reversed(range(E)):
        k = CAPN[e]
        sc = jnp.where(avail, probs[:, e:e + 1], -1e9)

        def search(_, carry):
            lo, hi = carry
            th = lo + (hi - lo) * frac
            cnt = jnp.sum((sc >= th).astype(jnp.float32), axis=0,
                          keepdims=True)
            ok = cnt >= k
            lo2 = jnp.max(jnp.where(ok, th, lo))
            hi2 = jnp.min(jnp.where(ok, 2.0, th))
            return (jnp.maximum(lo, lo2), jnp.minimum(hi, hi2))

        lo, _ = lax.fori_loop(0, _SEARCH_ITERS, search,
                              (jnp.float32(-2e9), jnp.float32(1.5)))
        assigned = avail & (sc >= lo)
        eidx = jnp.where(assigned, e, eidx)
        avail = avail & (~assigned)

    rp = jnp.zeros((N, 1), jnp.float32)
    for e in range(E):
        rp = jnp.where(eidx == e, probs[:, e:e + 1], rp)

    oh = (eidx == lax.broadcasted_iota(jnp.int32, (1, E), 1)
          ).astype(jnp.bfloat16)
    tl = (lax.broadcasted_iota(jnp.int32, (N, N), 1)
          < lax.broadcasted_iota(jnp.int32, (N, N), 0)
          ).astype(jnp.bfloat16)
    cnt = lax.dot_general(tl, oh, (((1,), (0,)), ((), ())),
                          preferred_element_type=jnp.float32)
    slot = jnp.zeros((N, 1), jnp.int32)
    for e in range(E):
        slot = jnp.where(
            eidx == e,
            OFF_OF_EXPERT[e] + cnt[:, e:e + 1].astype(jnp.int32), slot)
    slot_out_ref[...] = slot

    sel = (slot == lax.broadcasted_iota(jnp.int32, (1, N), 1)
           ).astype(jnp.bfloat16)
    xs = lax.dot_general(sel, x.astype(jnp.bfloat16),
                         (((0,), (0,)), ((), ())),
                         preferred_element_type=jnp.float32)
    xs = xs.astype(jnp.bfloat16)
    rps = lax.dot_general(sel, rp.astype(jnp.bfloat16),
                          (((0,), (0,)), ((), ())),
                          preferred_element_type=jnp.float32)
    xs_ref[...] = xs
    rps_ref[...] = rps

    g1 = g1_ref[...]
    b1 = b1_ref[...]
    qkv_parts = []
    for start, n, dim in BUCKETS:
        mixed = dim == 64 and n > 512
        xb = xs[start:start + n, :].astype(jnp.float32)
        xn = _ln(xb, g1, b1)
        if mixed:
            tokd = _mixed_tokd(n, start)
            cm = lax.broadcasted_iota(jnp.int32, (1, D), 1) < tokd
            xn = jnp.where(cm, xn, 0.0)
        xt = xn[:, :dim].astype(jnp.bfloat16)
        wb = jnp.concatenate(
            [wqkv_ref[:dim, j * D:j * D + dim] for j in range(3)], axis=1)
        qkv = lax.dot_general(xt, wb, (((1,), (0,)), ((), ())),
                              preferred_element_type=jnp.float32)
        if mixed:
            col = lax.broadcasted_iota(jnp.int32, (1, 3 * dim), 1) & (dim - 1)
            qkv = jnp.where(col < tokd, qkv, 0.0)
        qkv_parts.append(qkv.astype(jnp.bfloat16))

    def head_qkv(h, part):
        segs = []
        for (start, n, dim), qkv_b in zip(BUCKETS, qkv_parts):
            if dim > h * DH:
                off = part * dim + h * DH
                segs.append(qkv_b[:, off:off + DH])
        return jnp.concatenate(segs, axis=0)

    o_heads = {}
    for h0, nh, nq in HEAD_GROUPS:
        for i in range(nh):
            h = h0 + i
            q = head_qkv(h, 0)
            kk = head_qkv(h, 1)
            v = head_qkv(h, 2)
            if nq == N:
                chunks = []
                for c in range(GRID):
                    qc = q[c * NBLK:(c + 1) * NBLK, :]
                    s = lax.dot_general(
                        qc, kk, (((1,), (1,)), ((), ())),
                        preferred_element_type=jnp.float32) * (DH ** -0.5)
                    m = jnp.max(s, axis=1, keepdims=True)
                    p = jnp.exp(s - m)
                    p = p / jnp.sum(p, axis=1, keepdims=True)
                    chunks.append(lax.dot_general(
                        p.astype(jnp.bfloat16), v, (((1,), (0,)), ((), ())),
                        preferred_element_type=jnp.float32
                    ).astype(jnp.bfloat16))
                o_heads[h] = jnp.concatenate(chunks, axis=0)
            else:
                s = lax.dot_general(
                    q, kk, (((1,), (1,)), ((), ())),
                    preferred_element_type=jnp.float32) * (DH ** -0.5)
                m = jnp.maximum(jnp.max(s, axis=1, keepdims=True), 0.0)
                p = jnp.exp(s - m)
                denom = (jnp.sum(p, axis=1, keepdims=True)
                         + (N - nq) * jnp.exp(-m))
                o_heads[h] = lax.dot_general(
                    (p / denom).astype(jnp.bfloat16), v,
                    (((1,), (0,)), ((), ())),
                    preferred_element_type=jnp.float32).astype(jnp.bfloat16)

    for bi, (start, n, dim) in enumerate(BUCKETS):
        ob = jnp.concatenate(
            [o_heads[h][start:start + n, :] for h in range(dim // DH)],
            axis=1)
        o_refs[bi][...] = jnp.pad(ob, ((0, _npad(n) - n), (0, 0)))


def _tail_body(x_ref, xs_ref, slot_ref, rps_ref, wp_ref, bp_ref, g2_ref,
               b2_ref, w1_ref, bf1_ref, w2_ref, bf2_ref, alpha_ref,
               o0_ref, o1_ref, o2_ref, o3_ref, o4_ref, out_ref):
    o_refs = [o0_ref, o1_ref, o2_ref, o3_ref, o4_ref]
    g2 = g2_ref[...]
    b2 = b2_ref[...]
    alpha = alpha_ref[0, 0]
    comb_parts = []
    for bi, (start, n, dim) in enumerate(BUCKETS):
        mixed = dim == 64 and n > 512
        ob = o_refs[bi][0:n, :]
        xb = xs_ref[start:start + n, :].astype(jnp.float32)
        pr = lax.dot_general(ob, wp_ref[:dim, :dim],
                             (((1,), (0,)), ((), ())),
                             preferred_element_type=jnp.float32
                             ) + bp_ref[:, :dim]
        if mixed:
            tokd = _mixed_tokd(n, start)
            cm = lax.broadcasted_iota(jnp.int32, (1, dim), 1) < tokd
            pr = jnp.where(cm, pr, 0.0)
        if dim < D:
            z = xb + jnp.concatenate(
                [pr, jnp.zeros((n, D - dim), jnp.float32)], axis=1)
        else:
            z = xb + pr
        zn = _ln(z, g2, b2)[:, :dim]
        if mixed:
            zn = jnp.where(cm, zn, 0.0)
        h = lax.dot_general(zn.astype(jnp.bfloat16), w1_ref[:dim, :4 * dim],
                            (((1,), (0,)), ((), ())),
                            preferred_element_type=jnp.float32
                            ) + bf1_ref[:, :4 * dim]
        h = jax.nn.gelu(h)
        if mixed:
            hcol = lax.broadcasted_iota(jnp.int32, (1, 4 * dim), 1)
            h = jnp.where(hcol < tokd * 4, h, 0.0)
        zp = lax.dot_general(h.astype(jnp.bfloat16), w2_ref[:4 * dim, :dim],
                             (((1,), (0,)), ((), ())),
                             preferred_element_type=jnp.float32
                             ) + bf2_ref[:, :dim]
        if mixed:
            zp = jnp.where(cm, zp, 0.0)
        gate = alpha * rps_ref[start:start + n, :] + 1.0
        comb = pr + gate * zp
        if dim < D:
            comb = jnp.concatenate(
                [comb, jnp.zeros((n, D - dim), jnp.float32)], axis=1)
        comb_parts.append(comb.astype(jnp.bfloat16))

    comb_s = jnp.concatenate(comb_parts, axis=0)
    for c in range(GRID):
        sl = slot_ref[c * NBLK:(c + 1) * NBLK, :]
        sel = (sl == lax.broadcasted_iota(jnp.int32, (1, N), 1)
               ).astype(jnp.bfloat16)
        ret = lax.dot_general(sel, comb_s, (((1,), (0,)), ((), ())),
                              preferred_element_type=jnp.float32)
        out_ref[c * NBLK:(c + 1) * NBLK, :] = (
            x_ref[c * NBLK:(c + 1) * NBLK, :] + ret)


def kernel(x, Wr, br, g1, b1, Wqkv, Wproj, bproj, g2, b2, W1, bf1, W2, bf2,
           alpha):
    f32 = jnp.float32
    bf = jnp.bfloat16
    x2 = x.reshape(N, D)

    outs_a = pl.pallas_call(
        _fwd_body,
        out_shape=[jax.ShapeDtypeStruct((N, D), bf),
                   jax.ShapeDtypeStruct((N, 1), jnp.int32),
                   jax.ShapeDtypeStruct((N, 1), f32)]
        + [jax.ShapeDtypeStruct((_npad(n), dim), bf)
           for _, n, dim in BUCKETS],
    )(x2, Wr, br.reshape(1, E), g1.reshape(1, D), b1.reshape(1, D),
      Wqkv.astype(bf))
    xs, slot, rps = outs_a[0], outs_a[1], outs_a[2]
    o_buckets = outs_a[3:]

    out = pl.pallas_call(
        _tail_body,
        out_shape=jax.ShapeDtypeStruct((N, D), f32),
    )(x2, xs, slot, rps, Wproj.astype(bf), bproj.reshape(1, D),
      g2.reshape(1, D), b2.reshape(1, D), W1.astype(bf),
      bf1.reshape(1, 4 * D), W2.astype(bf), bf2.reshape(1, D),
      alpha.reshape(1, 1), *o_buckets)

    return out.reshape(1, N, D)


# speedup vs baseline: 3.2419x; 1.0881x over previous
import functools

import numpy as np
import jax
import jax.numpy as jnp
from jax import lax
from jax.experimental import pallas as pl

N = 2048
D = 1024
H = 16
DH = 64
E = 8
CAPN = [512, 409, 307, 204, 204, 163, 143, 106]

_ORDER = list(range(E - 1, -1, -1))
_SIZES = [CAPN[e] for e in _ORDER]
_OFFS = np.concatenate([[0], np.cumsum(_SIZES)])
OFF_OF_EXPERT = {e: int(_OFFS[i]) for i, e in enumerate(_ORDER)}

BUCKETS = [
    (0, 106, 1024),
    (106, 143, 512),
    (249, 163, 256),
    (412, 204, 128),
    (616, 1432, 64),
]

HEAD_GROUPS = [
    (0, 1, 2048),
    (1, 1, 616),
    (2, 2, 412),
    (4, 4, 249),
    (8, 8, 106),
]

_SEARCH_ITERS = 12
_NSPLIT = 16
NBLK = 256
GRID = N // NBLK


def _npad(n):
    return -(-n // 8) * 8


def _mixed_tokd(n, start):
    s = lax.broadcasted_iota(jnp.int32, (n, 1), 0) + start
    tokd = jnp.full((n, 1), 8, jnp.int32)
    for e in (1, 2, 3):
        lo = OFF_OF_EXPERT[e]
        tokd = jnp.where(s < lo + CAPN[e],
                         jnp.where(s >= lo, 8 << e, tokd), tokd)
    return tokd


def _ln(x, g, b):
    m = jnp.mean(x, axis=1, keepdims=True)
    v = jnp.mean((x - m) ** 2, axis=1, keepdims=True)
    return (x - m) * lax.rsqrt(v + 1e-6) * g + b


def _fwd_body(x_ref, wr_ref, br_ref, g1_ref, b1_ref, wqkv_ref,
              xs_ref, slot_out_ref, rps_ref, *o_refs):
    x = x_ref[...]
    logits = lax.dot_general(
        x.astype(jnp.bfloat16), wr_ref[...].astype(jnp.bfloat16),
        (((1,), (0,)), ((), ())),
        preferred_element_type=jnp.float32) + br_ref[...]
    mx = jnp.max(logits, axis=1, keepdims=True)
    ex = jnp.exp(logits - mx)
    probs = ex / jnp.sum(ex, axis=1, keepdims=True)

    frac = ((lax.broadcasted_iota(jnp.int32, (1, _NSPLIT), 1)
             .astype(jnp.float32) + 1.0) / (_NSPLIT + 1.0))
    tl = (lax.broadcasted_iota(jnp.int32, (N, N), 1)
          < lax.broadcasted_iota(jnp.int32, (N, N), 0)
          ).astype(jnp.bfloat16)
    avail = jnp.ones((N, 1), jnp.bool_)
    eidx = jnp.zeros((N, 1), jnp.int32)
    for e in reversed(range(E)):
        k = CAPN[e]
        sc = jnp.where(avail, probs[:, e:e + 1], -1e9)

        def search(_, carry):
            lo, hi = carry
            th = lo + (hi - lo) * frac
            cnt = jnp.sum((sc >= th).astype(jnp.float32), axis=0,
                          keepdims=True)
            ok = cnt >= k
            lo2 = jnp.max(jnp.where(ok, th, lo))
            hi2 = jnp.min(jnp.where(ok, 2.0, th))
            return (jnp.maximum(lo, lo2), jnp.minimum(hi, hi2))

        lo, _ = lax.fori_loop(0, _SEARCH_ITERS, search,
                              (jnp.float32(-2e9), jnp.float32(1.5)))
        assigned = avail & (sc >= lo)
        eidx = jnp.where(assigned, e, eidx)
        avail = avail & (~assigned)

    rp = jnp.zeros((N, 1), jnp.float32)
    for e in range(E):
        rp = jnp.where(eidx == e, probs[:, e:e + 1], rp)

    oh = (eidx == lax.broadcasted_iota(jnp.int32, (1, E), 1)
          ).astype(jnp.bfloat16)
    cnt = lax.dot_general(tl, oh, (((1,), (0,)), ((), ())),
                          preferred_element_type=jnp.float32)
    slot = jnp.zeros((N, 1), jnp.int32)
    for e in range(E):
        slot = jnp.where(
            eidx == e,
            OFF_OF_EXPERT[e] + cnt[:, e:e + 1].astype(jnp.int32), slot)
    slot_out_ref[...] = slot

    sel = (slot == lax.broadcasted_iota(jnp.int32, (1, N), 1)
           ).astype(jnp.bfloat16)
    xs = lax.dot_general(sel, x.astype(jnp.bfloat16),
                         (((0,), (0,)), ((), ())),
                         preferred_element_type=jnp.float32)
    xs = xs.astype(jnp.bfloat16)
    rps = lax.dot_general(sel, rp.astype(jnp.bfloat16),
                          (((0,), (0,)), ((), ())),
                          preferred_element_type=jnp.float32)
    xs_ref[...] = xs
    rps_ref[...] = rps

    g1 = g1_ref[...]
    b1 = b1_ref[...]
    qkv_parts = []
    for start, n, dim in BUCKETS:
        mixed = dim == 64 and n > 512
        xb = xs[start:start + n, :].astype(jnp.float32)
        xn = _ln(xb, g1, b1)
        if mixed:
            tokd = _mixed_tokd(n, start)
            cm = lax.broadcasted_iota(jnp.int32, (1, D), 1) < tokd
            xn = jnp.where(cm, xn, 0.0)
        xt = xn[:, :dim].astype(jnp.bfloat16)
        wb = jnp.concatenate(
            [wqkv_ref[:dim, j * D:j * D + dim] for j in range(3)], axis=1)
        qkv = lax.dot_general(xt, wb, (((1,), (0,)), ((), ())),
                              preferred_element_type=jnp.float32)
        if mixed:
            col = lax.broadcasted_iota(jnp.int32, (1, 3 * dim), 1) & (dim - 1)
            qkv = jnp.where(col < tokd, qkv, 0.0)
        qkv_parts.append(qkv.astype(jnp.bfloat16))

    def head_qkv(h, part):
        segs = []
        for (start, n, dim), qkv_b in zip(BUCKETS, qkv_parts):
            if dim > h * DH:
                off = part * dim + h * DH
                segs.append(qkv_b[:, off:off + DH])
        return jnp.concatenate(segs, axis=0)

    o_heads = {}
    for h0, nh, nq in HEAD_GROUPS:
        for i in range(nh):
            h = h0 + i
            q = head_qkv(h, 0)
            kk = head_qkv(h, 1)
            v = head_qkv(h, 2)
            if nq == N:
                chunks = []
                for c in range(GRID):
                    qc = q[c * NBLK:(c + 1) * NBLK, :]
                    s = lax.dot_general(
                        qc, kk, (((1,), (1,)), ((), ())),
                        preferred_element_type=jnp.float32) * (DH ** -0.5)
                    m = jnp.max(s, axis=1, keepdims=True)
                    p = jnp.exp(s - m)
                    p = p / jnp.sum(p, axis=1, keepdims=True)
                    chunks.append(lax.dot_general(
                        p.astype(jnp.bfloat16), v, (((1,), (0,)), ((), ())),
                        preferred_element_type=jnp.float32
                    ).astype(jnp.bfloat16))
                o_heads[h] = jnp.concatenate(chunks, axis=0)
            else:
                s = lax.dot_general(
                    q, kk, (((1,), (1,)), ((), ())),
                    preferred_element_type=jnp.float32) * (DH ** -0.5)
                m = jnp.maximum(jnp.max(s, axis=1, keepdims=True), 0.0)
                p = jnp.exp(s - m)
                denom = (jnp.sum(p, axis=1, keepdims=True)
                         + (N - nq) * jnp.exp(-m))
                o_heads[h] = lax.dot_general(
                    (p / denom).astype(jnp.bfloat16), v,
                    (((1,), (0,)), ((), ())),
                    preferred_element_type=jnp.float32).astype(jnp.bfloat16)

    for bi, (start, n, dim) in enumerate(BUCKETS):
        ob = jnp.concatenate(
            [o_heads[h][start:start + n, :] for h in range(dim // DH)],
            axis=1)
        o_refs[bi][...] = jnp.pad(ob, ((0, _npad(n) - n), (0, 0)))


def _tail_body(x_ref, xs_ref, slot_ref, rps_ref, wp_ref, bp_ref, g2_ref,
               b2_ref, w1_ref, bf1_ref, w2_ref, bf2_ref, alpha_ref,
               o0_ref, o1_ref, o2_ref, o3_ref, o4_ref, out_ref):
    o_refs = [o0_ref, o1_ref, o2_ref, o3_ref, o4_ref]
    g2 = g2_ref[...]
    b2 = b2_ref[...]
    alpha = alpha_ref[0, 0]
    comb_parts = []
    for bi, (start, n, dim) in enumerate(BUCKETS):
        mixed = dim == 64 and n > 512
        ob = o_refs[bi][0:n, :]
        xb = xs_ref[start:start + n, :].astype(jnp.float32)
        pr = lax.dot_general(ob, wp_ref[:dim, :dim],
                             (((1,), (0,)), ((), ())),
                             preferred_element_type=jnp.float32
                             ) + bp_ref[:, :dim]
        if mixed:
            tokd = _mixed_tokd(n, start)
            cm = lax.broadcasted_iota(jnp.int32, (1, dim), 1) < tokd
            pr = jnp.where(cm, pr, 0.0)
        if dim < D:
            z = xb + jnp.concatenate(
                [pr, jnp.zeros((n, D - dim), jnp.float32)], axis=1)
        else:
            z = xb + pr
        zn = _ln(z, g2, b2)[:, :dim]
        if mixed:
            zn = jnp.where(cm, zn, 0.0)
        h = lax.dot_general(zn.astype(jnp.bfloat16), w1_ref[:dim, :4 * dim],
                            (((1,), (0,)), ((), ())),
                            preferred_element_type=jnp.float32
                            ) + bf1_ref[:, :4 * dim]
        h = jax.nn.gelu(h)
        if mixed:
            hcol = lax.broadcasted_iota(jnp.int32, (1, 4 * dim), 1)
            h = jnp.where(hcol < tokd * 4, h, 0.0)
        zp = lax.dot_general(h.astype(jnp.bfloat16), w2_ref[:4 * dim, :dim],
                             (((1,), (0,)), ((), ())),
                             preferred_element_type=jnp.float32
                             ) + bf2_ref[:, :dim]
        if mixed:
            zp = jnp.where(cm, zp, 0.0)
        gate = alpha * rps_ref[start:start + n, :] + 1.0
        comb = pr + gate * zp
        if dim < D:
            comb = jnp.concatenate(
                [comb, jnp.zeros((n, D - dim), jnp.float32)], axis=1)
        comb_parts.append(comb.astype(jnp.bfloat16))

    comb_s = jnp.concatenate(comb_parts, axis=0)
    for c in range(GRID):
        sl = slot_ref[c * NBLK:(c + 1) * NBLK, :]
        sel = (sl == lax.broadcasted_iota(jnp.int32, (1, N), 1)
               ).astype(jnp.bfloat16)
        ret = lax.dot_general(sel, comb_s, (((1,), (0,)), ((), ())),
                              preferred_element_type=jnp.float32)
        out_ref[c * NBLK:(c + 1) * NBLK, :] = (
            x_ref[c * NBLK:(c + 1) * NBLK, :] + ret)


def kernel(x, Wr, br, g1, b1, Wqkv, Wproj, bproj, g2, b2, W1, bf1, W2, bf2,
           alpha):
    f32 = jnp.float32
    bf = jnp.bfloat16
    x2 = x.reshape(N, D)

    outs_a = pl.pallas_call(
        _fwd_body,
        out_shape=[jax.ShapeDtypeStruct((N, D), bf),
                   jax.ShapeDtypeStruct((N, 1), jnp.int32),
                   jax.ShapeDtypeStruct((N, 1), f32)]
        + [jax.ShapeDtypeStruct((_npad(n), dim), bf)
           for _, n, dim in BUCKETS],
    )(x2, Wr, br.reshape(1, E), g1.reshape(1, D), b1.reshape(1, D),
      Wqkv.astype(bf))
    xs, slot, rps = outs_a[0], outs_a[1], outs_a[2]
    o_buckets = outs_a[3:]

    out = pl.pallas_call(
        _tail_body,
        out_shape=jax.ShapeDtypeStruct((N, D), f32),
    )(x2, xs, slot, rps, Wproj.astype(bf), bproj.reshape(1, D),
      g2.reshape(1, D), b2.reshape(1, D), W1.astype(bf),
      bf1.reshape(1, 4 * D), W2.astype(bf), bf2.reshape(1, D),
      alpha.reshape(1, 1), *o_buckets)

    return out.reshape(1, N, D)
